# Initial kernel scaffold; baseline (speedup 1.0000x reference)
import jax
import jax.numpy as jnp
from jax.experimental import pallas as pl


def kernel(ent_output, W, b):
    raise NotImplementedError("write your pallas kernel here")


# speedup vs baseline: 1.1158x; 1.1158x over previous
import functools

import jax
import jax.numpy as jnp
from jax.experimental import pallas as pl

_ENT_DIM = 2048
_EXPANSION = 16
_K = 512
_IN_DIM = _ENT_DIM + 1
_OUT_DIM = _ENT_DIM * _EXPANSION
_BATCH = 16

_TILE_N = 2048
_NT = _OUT_DIM // _TILE_N


def _fused_kernel(x_ref, w_ref, b_ref, o_ref):
    i = pl.program_id(0)
    acc = jax.lax.dot_general(
        x_ref[...], w_ref[...],
        dimension_numbers=(((1,), (1,)), ((), ())),
        preferred_element_type=jnp.float32,
        precision=jax.lax.Precision.DEFAULT,
    )
    acc = jnp.maximum(acc + b_ref[...], 0.0)
    o_ref[:, pl.ds(i * _TILE_N, _TILE_N)] = acc

    @pl.when(i == _NT - 1)
    def _finalize():
        x = o_ref[...]
        xi = jax.lax.bitcast_convert_type(x, jnp.int32)

        def body(j, t):
            cand = t | (1 << (30 - j))
            cnt = jnp.sum((xi >= cand).astype(jnp.int32), axis=1, keepdims=True)
            return jnp.where(cnt >= _K, cand, t)

        t = jax.lax.fori_loop(0, 31, body, jnp.zeros((_BATCH, 1), jnp.int32))
        o_ref[...] = jnp.where(xi >= t, x, 0.0)


@jax.jit
def kernel(ent_output, W, b):
    b2 = b.reshape(1, _OUT_DIM)
    return pl.pallas_call(
        _fused_kernel,
        grid=(_NT,),
        in_specs=[
            pl.BlockSpec((_BATCH, _IN_DIM), lambda i: (0, 0)),
            pl.BlockSpec((_TILE_N, _IN_DIM), lambda i: (i, 0)),
            pl.BlockSpec((1, _TILE_N), lambda i: (0, i)),
        ],
        out_specs=pl.BlockSpec((_BATCH, _OUT_DIM), lambda i: (0, 0)),
        out_shape=jax.ShapeDtypeStruct((_BATCH, _OUT_DIM), jnp.float32),
    )(ent_output, W, b2)
